# baseline (speedup 1.0000x reference)
import functools

import jax
import jax.numpy as jnp
from jax import lax
from jax.experimental import pallas as pl
from jax.experimental.pallas import tpu as pltpu
from jax.experimental.pallas import tpu_sc as plsc

_EPS = 1e-5
_NC = 2
_NS = 16
_NW = _NC * _NS
_CHUNK = 128


def _bf16_bits(x):
    u = lax.bitcast_convert_type(x, jnp.uint32)
    lsb = (u >> 16) & jnp.uint32(1)
    return (u + jnp.uint32(0x7FFF) + lsb) >> 16


def _ln_pack(table_ref, gamma_ref, beta_ref, out_ref):
    t = table_ref[...]
    mean = jnp.mean(t, axis=1, keepdims=True)
    cen = t - mean
    var = jnp.mean(cen * cen, axis=1, keepdims=True)
    normed = cen * lax.rsqrt(var + _EPS) * gamma_ref[...] + beta_ref[...]
    ha = _bf16_bits(normed[:, :64])
    hb = _bf16_bits(normed[:, 64:])
    out_ref[...] = lax.bitcast_convert_type((hb << 16) | ha, jnp.int32)


def _unpack(in_ref, out_ref):
    w = lax.bitcast_convert_type(in_ref[...], jnp.uint32)
    a = lax.bitcast_convert_type(w << 16, jnp.float32)
    b = lax.bitcast_convert_type(w & jnp.uint32(0xFFFF0000), jnp.float32)
    out_ref[...] = jnp.concatenate([a, b], axis=1)


def kernel(task_id, batch_size, table, gamma, beta):
    V, D = table.shape
    B = task_id.shape[0]
    H = D // 2

    packed_tab = pl.pallas_call(
        _ln_pack,
        out_shape=jax.ShapeDtypeStruct((V, H), jnp.int32),
    )(table, gamma.reshape(1, D), beta.reshape(1, D))

    rows_per_w = B // _NW
    n_chunks = rows_per_w // _CHUNK
    idx2d = task_id.astype(jnp.int32).reshape(B // _CHUNK, _CHUNK)

    mesh = plsc.VectorSubcoreMesh(core_axis_name="c", subcore_axis_name="s")

    @functools.partial(
        pl.kernel,
        mesh=mesh,
        out_type=jax.ShapeDtypeStruct((B, H), jnp.int32),
        compiler_params=pltpu.CompilerParams(use_tc_tiling_on_sc=False),
        scratch_types=[
            pltpu.VMEM((n_chunks, _CHUNK), jnp.int32),
            pltpu.VMEM((rows_per_w, H), jnp.int32),
            pltpu.SemaphoreType.DMA,
        ],
    )
    def _gather(idx_hbm, tab_hbm, out_hbm, idx_v, rows_v, sem):
        wid = lax.axis_index("s") * _NC + lax.axis_index("c")
        pltpu.sync_copy(idx_hbm.at[pl.ds(wid * n_chunks, n_chunks)], idx_v)
        copies = [
            pltpu.async_copy(
                tab_hbm.at[idx_v.at[c]],
                rows_v.at[pl.ds(c * _CHUNK, _CHUNK)],
                sem,
            )
            for c in range(n_chunks)
        ]
        for cp in copies:
            cp.wait()
        pltpu.sync_copy(rows_v, out_hbm.at[pl.ds(wid * rows_per_w, rows_per_w)])

    packed_out = _gather(idx2d, packed_tab)

    blk = 2048
    return pl.pallas_call(
        _unpack,
        grid=(B // blk,),
        in_specs=[pl.BlockSpec((blk, H), lambda i: (i, 0))],
        out_specs=pl.BlockSpec((blk, D), lambda i: (i, 0)),
        out_shape=jax.ShapeDtypeStruct((B, D), jnp.float32),
    )(packed_out)


# speedup vs baseline: 1.6851x; 1.6851x over previous
import functools

import jax
import jax.numpy as jnp
from jax import lax
from jax.experimental import pallas as pl
from jax.experimental.pallas import tpu as pltpu
from jax.experimental.pallas import tpu_sc as plsc

_EPS = 1e-5
_NC = 2
_NS = 16
_NW = _NC * _NS
_CHUNK = 128
_STAGE_TILES = 8


def _ln_table(table_ref, gamma_ref, beta_ref, out_ref):
    t = table_ref[...]
    mean = jnp.mean(t, axis=1, keepdims=True)
    cen = t - mean
    var = jnp.mean(cen * cen, axis=1, keepdims=True)
    out_ref[...] = cen * lax.rsqrt(var + _EPS) * gamma_ref[...] + beta_ref[...]


def kernel(task_id, batch_size, table, gamma, beta):
    V, D = table.shape
    B = task_id.shape[0]

    normed = pl.pallas_call(
        _ln_table,
        out_shape=jax.ShapeDtypeStruct((V, D), jnp.float32),
    )(table, gamma.reshape(1, D), beta.reshape(1, D))

    rows_per_w = B // _NW
    n_chunks = rows_per_w // _CHUNK
    stage_rows = 128
    idx2d = task_id.astype(jnp.int32).reshape(B // _CHUNK, _CHUNK)

    mesh = plsc.VectorSubcoreMesh(core_axis_name="c", subcore_axis_name="s")

    @functools.partial(
        pl.kernel,
        mesh=mesh,
        out_type=jax.ShapeDtypeStruct((B, D), jnp.float32),
        scratch_types=[
            pltpu.VMEM((n_chunks, _CHUNK), jnp.int32),
            pltpu.VMEM((rows_per_w, D), jnp.float32),
            pltpu.VMEM_SHARED((V, D), jnp.float32),
            pltpu.SemaphoreType.DMA,
        ],
    )
    def _gather(idx_hbm, tab_hbm, out_hbm, idx_v, rows_v, shared_tab, sem):
        sid = lax.axis_index("s")
        wid = sid * _NC + lax.axis_index("c")
        pltpu.sync_copy(idx_hbm.at[pl.ds(wid * n_chunks, n_chunks)], idx_v)

        @pl.when(sid < _STAGE_TILES - 1)
        def _stage():
            r0 = sid * stage_rows
            pltpu.sync_copy(
                tab_hbm.at[pl.ds(r0, stage_rows)],
                shared_tab.at[pl.ds(r0, stage_rows)],
            )

        @pl.when(sid == _STAGE_TILES - 1)
        def _stage_tail():
            r0 = (_STAGE_TILES - 1) * stage_rows
            pltpu.sync_copy(
                tab_hbm.at[pl.ds(r0, V - r0)],
                shared_tab.at[pl.ds(r0, V - r0)],
            )

        plsc.subcore_barrier()

        copies = [
            pltpu.async_copy(
                shared_tab.at[idx_v.at[c]],
                rows_v.at[pl.ds(c * _CHUNK, _CHUNK)],
                sem,
            )
            for c in range(n_chunks)
        ]
        for cp in copies:
            cp.wait()
        pltpu.sync_copy(rows_v, out_hbm.at[pl.ds(wid * rows_per_w, rows_per_w)])

    return _gather(idx2d, normed)


# speedup vs baseline: 1.7398x; 1.0325x over previous
import functools

import jax
import jax.numpy as jnp
from jax import lax
from jax.experimental import pallas as pl
from jax.experimental.pallas import tpu as pltpu
from jax.experimental.pallas import tpu_sc as plsc

_EPS = 1e-5
_NC = 2
_NS = 16
_NW = _NC * _NS
_CHUNK = 128
_STAGE_TILES = 8


def _ln_table(table_ref, gamma_ref, beta_ref, out_ref):
    t = table_ref[...]
    mean = jnp.mean(t, axis=1, keepdims=True)
    cen = t - mean
    var = jnp.mean(cen * cen, axis=1, keepdims=True)
    out_ref[...] = cen * lax.rsqrt(var + _EPS) * gamma_ref[...] + beta_ref[...]


def kernel(task_id, batch_size, table, gamma, beta):
    V, D = table.shape
    B = task_id.shape[0]

    normed = pl.pallas_call(
        _ln_table,
        out_shape=jax.ShapeDtypeStruct((V, D), jnp.float32),
    )(table, gamma.reshape(1, D), beta.reshape(1, D))

    rows_per_w = B // _NW
    n_chunks = rows_per_w // _CHUNK
    stage_rows = 128
    idx2d = task_id.astype(jnp.int32).reshape(B // _CHUNK, _CHUNK)

    mesh = plsc.VectorSubcoreMesh(core_axis_name="c", subcore_axis_name="s")

    @functools.partial(
        pl.kernel,
        mesh=mesh,
        out_type=jax.ShapeDtypeStruct((B, D), jnp.float32),
        scratch_types=[
            pltpu.VMEM((n_chunks, _CHUNK), jnp.int32),
            pltpu.VMEM((rows_per_w, D), jnp.float32),
            pltpu.VMEM_SHARED((V, D), jnp.float32),
            pltpu.SemaphoreType.DMA,
            pltpu.SemaphoreType.DMA,
            pltpu.SemaphoreType.DMA,
            pltpu.SemaphoreType.DMA,
        ],
    )
    def _gather(idx_hbm, tab_hbm, out_hbm, idx_v, rows_v, shared_tab,
                g0, g1, w0, w1):
        sid = lax.axis_index("s")
        wid = sid * _NC + lax.axis_index("c")
        pltpu.sync_copy(idx_hbm.at[pl.ds(wid * n_chunks, n_chunks)], idx_v)

        @pl.when(sid < _STAGE_TILES - 1)
        def _stage():
            r0 = sid * stage_rows
            pltpu.sync_copy(
                tab_hbm.at[pl.ds(r0, stage_rows)],
                shared_tab.at[pl.ds(r0, stage_rows)],
            )

        @pl.when(sid == _STAGE_TILES - 1)
        def _stage_tail():
            r0 = (_STAGE_TILES - 1) * stage_rows
            pltpu.sync_copy(
                tab_hbm.at[pl.ds(r0, V - r0)],
                shared_tab.at[pl.ds(r0, V - r0)],
            )

        plsc.subcore_barrier()

        base = wid * rows_per_w
        gsem = (g0, g1)
        wsem = (w0, w1)

        def fire_gather(c):
            return pltpu.async_copy(
                shared_tab.at[idx_v.at[c]],
                rows_v.at[pl.ds(c * _CHUNK, _CHUNK)],
                gsem[c % 2],
            )

        def fire_wb(c):
            return pltpu.async_copy(
                rows_v.at[pl.ds(c * _CHUNK, _CHUNK)],
                out_hbm.at[pl.ds(base + c * _CHUNK, _CHUNK)],
                wsem[c % 2],
            )

        gathers = [fire_gather(0), fire_gather(1)] + [None] * (n_chunks - 2)
        wbs = [None] * n_chunks
        for c in range(n_chunks):
            gathers[c].wait()
            if c + 2 < n_chunks:
                gathers[c + 2] = fire_gather(c + 2)
            if c - 2 >= 0:
                wbs[c - 2].wait()
            wbs[c] = fire_wb(c)
        for c in range(max(n_chunks - 2, 0), n_chunks):
            wbs[c].wait()

    return _gather(idx2d, normed)
